# Initial kernel scaffold; baseline (speedup 1.0000x reference)
import jax
import jax.numpy as jnp
from jax.experimental import pallas as pl


def kernel(feats1, points1, feats2, points2, W1, b1, g1, beta1, W2, b2, g2, beta2):
    raise NotImplementedError("write your pallas kernel here")


# speedup vs baseline: 29.4811x; 29.4811x over previous
import functools

import jax
import jax.numpy as jnp
from jax.experimental import pallas as pl

_EPS = 1e-08
_M = 256


def _stats1_kernel(x_ref, w_ref, b_ref, acc_ref):
    i = pl.program_id(0)
    y = jnp.dot(x_ref[...], w_ref[...].T, preferred_element_type=jnp.float32)
    y = y + b_ref[...]
    s = jnp.sum(y, axis=0, keepdims=True)
    ss = jnp.sum(y * y, axis=0, keepdims=True)
    blk = jnp.concatenate([s, ss], axis=0)

    @pl.when(i == 0)
    def _():
        acc_ref[...] = blk

    @pl.when(i > 0)
    def _():
        acc_ref[...] += blk


def _f2_kernel(x_ref, w_ref, b_ref, g_ref, beta_ref, out_ref):
    y = jnp.dot(x_ref[...], w_ref[...].T, preferred_element_type=jnp.float32)
    y = y + b_ref[...]
    m = jnp.mean(y, axis=0, keepdims=True)
    v = jnp.mean((y - m) ** 2, axis=0, keepdims=True)
    out_ref[...] = jnp.maximum(
        (y - m) / jnp.sqrt(v + 1e-5) * g_ref[...] + beta_ref[...], 0.0
    )


def _main_kernel(x1_ref, p1_ref, p2t_ref, f2_ref, w1_ref, b1_ref, g1_ref,
                 beta1_ref, stats_ref, out_ref, *, n_rows, n2):
    mean = stats_ref[0:1, :] / n_rows
    var = stats_ref[1:2, :] / n_rows - mean * mean
    x1 = x1_ref[0]
    y1 = jnp.dot(x1, w1_ref[...].T, preferred_element_type=jnp.float32)
    y1 = y1 + b1_ref[...]
    f1 = jnp.maximum(
        (y1 - mean) / jnp.sqrt(var + 1e-5) * g1_ref[...] + beta1_ref[...], 0.0
    )

    p1 = p1_ref[0]
    p2t = p2t_ref[0]
    q2 = jnp.sum(p1 * p1, axis=1, keepdims=True)
    r2 = jnp.sum(p2t * p2t, axis=0, keepdims=True)
    cross = jnp.dot(p1, p2t, preferred_element_type=jnp.float32)
    dist = jnp.sqrt(jnp.maximum(q2 + r2 - 2.0 * cross, 0.0))

    iota = jax.lax.broadcasted_iota(jnp.int32, dist.shape, 1)
    d = dist
    recips = []
    sels = []
    for _ in range(3):
        mk = jnp.min(d, axis=1, keepdims=True)
        idxk = jnp.min(jnp.where(d == mk, iota, n2), axis=1, keepdims=True)
        sel = iota == idxk
        recips.append(1.0 / (mk + _EPS))
        sels.append(sel)
        d = jnp.where(sel, jnp.float32(1e30), d)
    norm = recips[0] + recips[1] + recips[2]
    oh = jnp.where(sels[0], recips[0] / norm, 0.0)
    oh = oh + jnp.where(sels[1], recips[1] / norm, 0.0)
    oh = oh + jnp.where(sels[2], recips[2] / norm, 0.0)
    new = jnp.dot(oh, f2_ref[0], preferred_element_type=jnp.float32)
    out_ref[0] = f1 + new


def kernel(feats1, points1, feats2, points2, W1, b1, g1, beta1, W2, b2, g2,
           beta2):
    B, N1, C1 = feats1.shape
    _, N2, C2 = feats2.shape
    C = W1.shape[0]
    x1 = feats1.reshape(B * N1, C1)
    x2 = feats2.reshape(B * N2, C2)
    b1r = b1.reshape(1, C)
    g1r = g1.reshape(1, C)
    beta1r = beta1.reshape(1, C)
    b2r = b2.reshape(1, C)
    g2r = g2.reshape(1, C)
    beta2r = beta2.reshape(1, C)

    chunk = 4096
    nchunks = (B * N1) // chunk
    stats = pl.pallas_call(
        _stats1_kernel,
        grid=(nchunks,),
        in_specs=[
            pl.BlockSpec((chunk, C1), lambda i: (i, 0)),
            pl.BlockSpec((C, C1), lambda i: (0, 0)),
            pl.BlockSpec((1, C), lambda i: (0, 0)),
        ],
        out_specs=pl.BlockSpec((2, C), lambda i: (0, 0)),
        out_shape=jax.ShapeDtypeStruct((2, C), jnp.float32),
    )(x1, W1, b1r)

    f2 = pl.pallas_call(
        _f2_kernel,
        out_shape=jax.ShapeDtypeStruct((B * N2, C), jnp.float32),
    )(x2, W2, b2r, g2r, beta2r)
    f2 = f2.reshape(B, N2, C)

    p2t = points2.transpose(0, 2, 1)

    nb = N1 // _M
    out = pl.pallas_call(
        functools.partial(_main_kernel, n_rows=float(B * N1), n2=N2),
        grid=(B, nb),
        in_specs=[
            pl.BlockSpec((1, _M, C1), lambda b, n: (b, n, 0)),
            pl.BlockSpec((1, _M, 3), lambda b, n: (b, n, 0)),
            pl.BlockSpec((1, 3, N2), lambda b, n: (b, 0, 0)),
            pl.BlockSpec((1, N2, C), lambda b, n: (b, 0, 0)),
            pl.BlockSpec((C, C1), lambda b, n: (0, 0)),
            pl.BlockSpec((1, C), lambda b, n: (0, 0)),
            pl.BlockSpec((1, C), lambda b, n: (0, 0)),
            pl.BlockSpec((1, C), lambda b, n: (0, 0)),
            pl.BlockSpec((2, C), lambda b, n: (0, 0)),
        ],
        out_specs=pl.BlockSpec((1, _M, C), lambda b, n: (b, n, 0)),
        out_shape=jax.ShapeDtypeStruct((B, N1, C), jnp.float32),
    )(feats1, points1, p2t, f2, W1, b1r, g1r, beta1r, stats)

    return (out, points1)


# speedup vs baseline: 50.4431x; 1.7110x over previous
import functools

import jax
import jax.numpy as jnp
from jax.experimental import pallas as pl

_EPS = 1e-08
_M = 256


def _stats1_kernel(x_ref, w_ref, b_ref, acc_ref):
    i = pl.program_id(0)
    y = jnp.dot(x_ref[...], w_ref[...].T, preferred_element_type=jnp.float32)
    y = y + b_ref[...]
    s = jnp.sum(y, axis=0, keepdims=True)
    ss = jnp.sum(y * y, axis=0, keepdims=True)
    blk = jnp.concatenate([s, ss], axis=0)

    @pl.when(i == 0)
    def _():
        acc_ref[...] = blk

    @pl.when(i > 0)
    def _():
        acc_ref[...] += blk


def _f2_kernel(x_ref, w_ref, b_ref, g_ref, beta_ref, out_ref):
    y = jnp.dot(x_ref[...], w_ref[...].T, preferred_element_type=jnp.float32)
    y = y + b_ref[...]
    m = jnp.mean(y, axis=0, keepdims=True)
    v = jnp.mean((y - m) ** 2, axis=0, keepdims=True)
    out_ref[...] = jnp.maximum(
        (y - m) / jnp.sqrt(v + 1e-5) * g_ref[...] + beta_ref[...], 0.0
    )


def _main_kernel(x1_ref, p1_ref, p2t_ref, f2_ref, w1_ref, b1_ref, g1_ref,
                 beta1_ref, stats_ref, out_ref, *, n_rows, n2):
    mean = stats_ref[0:1, :] / n_rows
    var = stats_ref[1:2, :] / n_rows - mean * mean
    x1 = x1_ref[0]
    y1 = jnp.dot(x1, w1_ref[...].T, preferred_element_type=jnp.float32)
    y1 = y1 + b1_ref[...]
    f1 = jnp.maximum(
        (y1 - mean) / jnp.sqrt(var + 1e-5) * g1_ref[...] + beta1_ref[...], 0.0
    )

    p1 = p1_ref[0]
    p2t = p2t_ref[0]
    q2 = jnp.sum(p1 * p1, axis=1, keepdims=True)
    r2 = jnp.sum(p2t * p2t, axis=0, keepdims=True)
    cross = jnp.dot(p1, p2t, preferred_element_type=jnp.float32)
    d2 = jnp.maximum(q2 + r2 - 2.0 * cross, 0.0)

    big = jnp.float32(3e38)
    m1 = jnp.min(d2, axis=1, keepdims=True)
    d2b = jnp.where(d2 == m1, big, d2)
    m2 = jnp.min(d2b, axis=1, keepdims=True)
    m3 = jnp.min(jnp.where(d2b == m2, big, d2b), axis=1, keepdims=True)
    r1 = 1.0 / (jnp.sqrt(m1) + _EPS)
    r2w = 1.0 / (jnp.sqrt(m2) + _EPS)
    r3 = 1.0 / (jnp.sqrt(m3) + _EPS)
    norm = r1 + r2w + r3
    oh = jnp.where(
        d2 == m1, r1 / norm,
        jnp.where(d2 == m2, r2w / norm,
                  jnp.where(d2 == m3, r3 / norm, 0.0)),
    )
    new = jnp.dot(oh, f2_ref[0], preferred_element_type=jnp.float32)
    out_ref[0] = f1 + new


def kernel(feats1, points1, feats2, points2, W1, b1, g1, beta1, W2, b2, g2,
           beta2):
    B, N1, C1 = feats1.shape
    _, N2, C2 = feats2.shape
    C = W1.shape[0]
    x1 = feats1.reshape(B * N1, C1)
    x2 = feats2.reshape(B * N2, C2)
    b1r = b1.reshape(1, C)
    g1r = g1.reshape(1, C)
    beta1r = beta1.reshape(1, C)
    b2r = b2.reshape(1, C)
    g2r = g2.reshape(1, C)
    beta2r = beta2.reshape(1, C)

    chunk = 4096
    nchunks = (B * N1) // chunk
    stats = pl.pallas_call(
        _stats1_kernel,
        grid=(nchunks,),
        in_specs=[
            pl.BlockSpec((chunk, C1), lambda i: (i, 0)),
            pl.BlockSpec((C, C1), lambda i: (0, 0)),
            pl.BlockSpec((1, C), lambda i: (0, 0)),
        ],
        out_specs=pl.BlockSpec((2, C), lambda i: (0, 0)),
        out_shape=jax.ShapeDtypeStruct((2, C), jnp.float32),
    )(x1, W1, b1r)

    f2 = pl.pallas_call(
        _f2_kernel,
        out_shape=jax.ShapeDtypeStruct((B * N2, C), jnp.float32),
    )(x2, W2, b2r, g2r, beta2r)
    f2 = f2.reshape(B, N2, C)

    p2t = points2.transpose(0, 2, 1)

    nb = N1 // _M
    out = pl.pallas_call(
        functools.partial(_main_kernel, n_rows=float(B * N1), n2=N2),
        grid=(B, nb),
        in_specs=[
            pl.BlockSpec((1, _M, C1), lambda b, n: (b, n, 0)),
            pl.BlockSpec((1, _M, 3), lambda b, n: (b, n, 0)),
            pl.BlockSpec((1, 3, N2), lambda b, n: (b, 0, 0)),
            pl.BlockSpec((1, N2, C), lambda b, n: (b, 0, 0)),
            pl.BlockSpec((C, C1), lambda b, n: (0, 0)),
            pl.BlockSpec((1, C), lambda b, n: (0, 0)),
            pl.BlockSpec((1, C), lambda b, n: (0, 0)),
            pl.BlockSpec((1, C), lambda b, n: (0, 0)),
            pl.BlockSpec((2, C), lambda b, n: (0, 0)),
        ],
        out_specs=pl.BlockSpec((1, _M, C), lambda b, n: (b, n, 0)),
        out_shape=jax.ShapeDtypeStruct((B, N1, C), jnp.float32),
    )(feats1, points1, p2t, f2, W1, b1r, g1r, beta1r, stats)

    return (out, points1)
